# baseline (speedup 1.0000x reference)
import functools

import jax
import jax.numpy as jnp
from jax import lax
from jax.experimental import pallas as pl
from jax.experimental.pallas import tpu as pltpu
from jax.experimental.pallas import tpu_sc as plsc

N_DST = 10000
D = 128
NC = 2
NS = 16
NW = NC * NS
CHUNK = 80
M = 128
E_PAD = NW * M * CHUNK
ACC_ROWS = 10240
ROWS_PER_TILE = ACC_ROWS // NS


def _sc_partial_sums(node, src2, dst3, zeros):
    mesh = plsc.VectorSubcoreMesh(core_axis_name="c", subcore_axis_name="s")

    @functools.partial(
        pl.kernel,
        mesh=mesh,
        out_type=jax.ShapeDtypeStruct((NC * ACC_ROWS, D), jnp.float32),
        scratch_types=[
            pltpu.VMEM((M * CHUNK,), jnp.int32),
            pltpu.VMEM((M, CHUNK), jnp.int32),
            pltpu.VMEM((CHUNK, D), jnp.float32),
            pltpu.VMEM_SHARED((ACC_ROWS, D), jnp.float32),
            pltpu.SemaphoreType.DMA,
        ],
    )
    def k(node_hbm, src_hbm, dst_hbm, zeros_hbm, out_hbm,
          src_v, dst_v, rows_v, acc, sem):
        c = lax.axis_index("c")
        s = lax.axis_index("s")
        wid = s * NC + c

        r0 = s * ROWS_PER_TILE
        pltpu.sync_copy(zeros_hbm.at[pl.ds(r0, ROWS_PER_TILE)],
                        acc.at[pl.ds(r0, ROWS_PER_TILE)])

        pltpu.sync_copy(src_hbm.at[wid], src_v)
        pltpu.sync_copy(dst_hbm.at[wid], dst_v)
        plsc.subcore_barrier()

        def body(chunk, carry):
            idx = src_v.at[pl.ds(chunk * CHUNK, CHUNK)]
            pltpu.async_copy(node_hbm.at[idx], rows_v, sem).wait()
            pltpu.sync_copy(rows_v, acc.at[dst_v.at[chunk]], add=True)
            return carry

        lax.fori_loop(0, M, body, 0)
        plsc.subcore_barrier()

        pltpu.sync_copy(acc.at[pl.ds(r0, ROWS_PER_TILE)],
                        out_hbm.at[pl.ds(c * ACC_ROWS + r0, ROWS_PER_TILE)])

    return k(node, src2, dst3, zeros)


def _combine(partials):
    R = 400

    def body(a_ref, b_ref, o_ref):
        o_ref[...] = a_ref[...] + b_ref[...]

    return pl.pallas_call(
        body,
        grid=(N_DST // R,),
        in_specs=[pl.BlockSpec((R, D), lambda i: (i, 0)),
                  pl.BlockSpec((R, D), lambda i: (i, 0))],
        out_specs=pl.BlockSpec((R, D), lambda i: (i, 0)),
        out_shape=jax.ShapeDtypeStruct((N_DST, D), jnp.float32),
    )(partials[:N_DST], partials[ACC_ROWS:ACC_ROWS + N_DST])


def kernel(node, edge_index):
    ei = edge_index.astype(jnp.int32)
    E = ei.shape[1]
    pad = E_PAD - E
    src = jnp.concatenate([ei[0], jnp.zeros((pad,), jnp.int32)])
    dst = jnp.concatenate([ei[1], jnp.full((pad,), N_DST, jnp.int32)])
    src2 = src.reshape(NW, M * CHUNK)
    dst3 = dst.reshape(NW, M, CHUNK)
    zeros = jnp.zeros((ACC_ROWS, D), jnp.float32)
    partials = _sc_partial_sums(node, src2, dst3, zeros)
    return _combine(partials)


# speedup vs baseline: 1.1466x; 1.1466x over previous
import functools

import jax
import jax.numpy as jnp
from jax import lax
from jax.experimental import pallas as pl
from jax.experimental.pallas import tpu as pltpu
from jax.experimental.pallas import tpu_sc as plsc

N_DST = 10000
D = 128
NC = 2
NS = 16
NW = NC * NS
CHUNK = 80
M = 128
E_PAD = NW * M * CHUNK
ACC_ROWS = 10240
ROWS_PER_TILE = ACC_ROWS // NS
L = 16


def _sc_partial_sums(node, src2, dst2, zeros):
    mesh = plsc.VectorSubcoreMesh(core_axis_name="c", subcore_axis_name="s")

    @functools.partial(
        pl.kernel,
        mesh=mesh,
        out_type=jax.ShapeDtypeStruct((NC * ACC_ROWS, D), jnp.float32),
        scratch_types=[
            pltpu.VMEM((M * CHUNK,), jnp.int32),
            pltpu.VMEM((M * CHUNK,), jnp.int32),
            pltpu.VMEM((CHUNK,), jnp.int32),
            pltpu.VMEM((CHUNK,), jnp.int32),
            pltpu.VMEM((CHUNK,), jnp.int32),
            pltpu.VMEM((CHUNK,), jnp.int32),
            pltpu.VMEM((CHUNK, D), jnp.float32),
            pltpu.VMEM((CHUNK, D), jnp.float32),
            pltpu.VMEM_SHARED((ACC_ROWS, D), jnp.float32),
            pltpu.SemaphoreType.DMA,
            pltpu.SemaphoreType.DMA,
            pltpu.SemaphoreType.DMA,
        ],
    )
    def k(node_hbm, src_hbm, dst_hbm, zeros_hbm, out_hbm,
          src_all, dst_all, src_a, dst_a, src_b, dst_b, rows_a, rows_b,
          acc, sem_a, sem_b, sem_s):
        c = lax.axis_index("c")
        s = lax.axis_index("s")
        wid = s * NC + c

        r0 = s * ROWS_PER_TILE
        pltpu.sync_copy(zeros_hbm.at[pl.ds(r0, ROWS_PER_TILE)],
                        acc.at[pl.ds(r0, ROWS_PER_TILE)])

        pltpu.sync_copy(src_hbm.at[wid], src_all)
        pltpu.sync_copy(dst_hbm.at[wid], dst_all)
        plsc.subcore_barrier()

        def regcopy(chunk, src_c, dst_c):
            off = chunk * CHUNK
            for kk in range(CHUNK // L):
                src_c[pl.ds(kk * L, L)] = src_all[pl.ds(off + kk * L, L)]
                dst_c[pl.ds(kk * L, L)] = dst_all[pl.ds(off + kk * L, L)]

        def gather_start(src_c, rows_v, sem):
            pltpu.async_copy(node_hbm.at[src_c], rows_v, sem)

        def gather_wait(src_c, rows_v, sem):
            pltpu.make_async_copy(node_hbm.at[src_c], rows_v, sem).wait()

        def scatter_sync(dst_c, rows_v):
            pltpu.async_copy(rows_v, acc.at[dst_c], sem_s, add=True).wait()

        regcopy(0, src_a, dst_a)
        gather_start(src_a, rows_a, sem_a)
        regcopy(1, src_b, dst_b)
        gather_start(src_b, rows_b, sem_b)

        def body(i, carry):
            gather_wait(src_a, rows_a, sem_a)
            scatter_sync(dst_a, rows_a)
            regcopy(2 * i + 2, src_a, dst_a)
            gather_start(src_a, rows_a, sem_a)
            gather_wait(src_b, rows_b, sem_b)
            scatter_sync(dst_b, rows_b)
            regcopy(2 * i + 3, src_b, dst_b)
            gather_start(src_b, rows_b, sem_b)
            return carry

        lax.fori_loop(0, M // 2 - 1, body, 0)
        gather_wait(src_a, rows_a, sem_a)
        scatter_sync(dst_a, rows_a)
        gather_wait(src_b, rows_b, sem_b)
        scatter_sync(dst_b, rows_b)
        plsc.subcore_barrier()

        pltpu.sync_copy(acc.at[pl.ds(r0, ROWS_PER_TILE)],
                        out_hbm.at[pl.ds(c * ACC_ROWS + r0, ROWS_PER_TILE)])

    return k(node, src2, dst2, zeros)


def _combine(partials):
    R = 400

    def body(a_ref, b_ref, o_ref):
        o_ref[...] = a_ref[...] + b_ref[...]

    return pl.pallas_call(
        body,
        grid=(N_DST // R,),
        in_specs=[pl.BlockSpec((R, D), lambda i: (i, 0)),
                  pl.BlockSpec((R, D), lambda i: (i, 0))],
        out_specs=pl.BlockSpec((R, D), lambda i: (i, 0)),
        out_shape=jax.ShapeDtypeStruct((N_DST, D), jnp.float32),
    )(partials[:N_DST], partials[ACC_ROWS:ACC_ROWS + N_DST])


def kernel(node, edge_index):
    ei = edge_index.astype(jnp.int32)
    E = ei.shape[1]
    pad = E_PAD - E
    src = jnp.concatenate([ei[0], jnp.zeros((pad,), jnp.int32)])
    dst = jnp.concatenate([ei[1], jnp.full((pad,), N_DST, jnp.int32)])
    src2 = src.reshape(NW, M * CHUNK)
    dst2 = dst.reshape(NW, M * CHUNK)
    zeros = jnp.zeros((ACC_ROWS, D), jnp.float32)
    partials = _sc_partial_sums(node, src2, dst2, zeros)
    return _combine(partials)


# speedup vs baseline: 2.6383x; 2.3008x over previous
import functools

import jax
import jax.numpy as jnp
from jax import lax
from jax.experimental import pallas as pl
from jax.experimental.pallas import tpu as pltpu
from jax.experimental.pallas import tpu_sc as plsc

N_DST = 10000
D = 128
NC = 2
NS = 16
NW = NC * NS
CHUNK = 80
ACC_ROWS = 10240
ROWS_PER_TILE = ACC_ROWS // NS


def _sc_partial_sums(node, src, dst, zeros):
    E = src.shape[0]
    per_tile = E // NW
    m = per_tile // CHUNK
    n_pairs = m // 2

    mesh = plsc.VectorSubcoreMesh(core_axis_name="c", subcore_axis_name="s")

    @functools.partial(
        pl.kernel,
        mesh=mesh,
        out_type=jax.ShapeDtypeStruct((NC * ACC_ROWS, D), jnp.float32),
        scratch_types=[
            pltpu.VMEM((CHUNK,), jnp.int32),
            pltpu.VMEM((CHUNK,), jnp.int32),
            pltpu.VMEM((CHUNK,), jnp.int32),
            pltpu.VMEM((CHUNK,), jnp.int32),
            pltpu.VMEM((CHUNK, D), jnp.float32),
            pltpu.VMEM((CHUNK, D), jnp.float32),
            pltpu.VMEM_SHARED((ACC_ROWS, D), jnp.float32),
            pltpu.SemaphoreType.DMA,
            pltpu.SemaphoreType.DMA,
            pltpu.SemaphoreType.DMA,
        ],
    )
    def k(node_hbm, src_hbm, dst_hbm, zeros_hbm, out_hbm,
          src_a, dst_a, src_b, dst_b, rows_a, rows_b,
          acc, sem_a, sem_b, sem_s):
        c = lax.axis_index("c")
        s = lax.axis_index("s")
        wid = s * NC + c

        r0 = s * ROWS_PER_TILE
        pltpu.sync_copy(zeros_hbm.at[pl.ds(r0, ROWS_PER_TILE)],
                        acc.at[pl.ds(r0, ROWS_PER_TILE)])
        plsc.subcore_barrier()

        base0 = wid * per_tile

        def idx_load(chunk, src_c, dst_c):
            e = base0 + chunk * CHUNK
            pltpu.sync_copy(src_hbm.at[pl.ds(e, CHUNK)], src_c)
            pltpu.sync_copy(dst_hbm.at[pl.ds(e, CHUNK)], dst_c)

        def gather_start(src_c, rows_v, sem):
            pltpu.async_copy(node_hbm.at[src_c], rows_v, sem)

        def gather_wait(src_c, rows_v, sem):
            pltpu.make_async_copy(node_hbm.at[src_c], rows_v, sem).wait()

        def scatter_sync(dst_c, rows_v):
            pltpu.sync_copy(rows_v, acc.at[dst_c], add=True)

        idx_load(0, src_a, dst_a)
        gather_start(src_a, rows_a, sem_a)

        def body(i, carry):
            idx_load(2 * i + 1, src_b, dst_b)
            gather_start(src_b, rows_b, sem_b)
            gather_wait(src_a, rows_a, sem_a)
            scatter_sync(dst_a, rows_a)
            idx_load(2 * i + 2, src_a, dst_a)
            gather_start(src_a, rows_a, sem_a)
            gather_wait(src_b, rows_b, sem_b)
            scatter_sync(dst_b, rows_b)
            return carry

        lax.fori_loop(0, n_pairs, body, 0)
        gather_wait(src_a, rows_a, sem_a)
        scatter_sync(dst_a, rows_a)
        plsc.subcore_barrier()

        pltpu.sync_copy(acc.at[pl.ds(r0, ROWS_PER_TILE)],
                        out_hbm.at[pl.ds(c * ACC_ROWS + r0, ROWS_PER_TILE)])

    return k(node, src, dst, zeros)


def _combine(partials):
    R = 400

    def body(a_ref, b_ref, o_ref):
        o_ref[...] = a_ref[...] + b_ref[...]

    return pl.pallas_call(
        body,
        grid=(N_DST // R,),
        in_specs=[pl.BlockSpec((R, D), lambda i: (i, 0)),
                  pl.BlockSpec((R, D), lambda i: (i, 0))],
        out_specs=pl.BlockSpec((R, D), lambda i: (i, 0)),
        out_shape=jax.ShapeDtypeStruct((N_DST, D), jnp.float32),
    )(partials[:N_DST], partials[ACC_ROWS:ACC_ROWS + N_DST])


def kernel(node, edge_index):
    ei = edge_index.astype(jnp.int32)
    zeros = jnp.zeros((ACC_ROWS, D), jnp.float32)
    partials = _sc_partial_sums(node, ei[0], ei[1], zeros)
    return _combine(partials)


# speedup vs baseline: 3.1248x; 1.1844x over previous
import functools

import jax
import jax.numpy as jnp
from jax import lax
from jax.experimental import pallas as pl
from jax.experimental.pallas import tpu as pltpu
from jax.experimental.pallas import tpu_sc as plsc

N_DST = 10000
D = 128
NC = 2
NS = 16
NW = NC * NS
CHUNK = 80
ACC_ROWS = 10240
ROWS_PER_TILE = ACC_ROWS // NS


def _sc_partial_sums(node, src, dst, zeros):
    E = src.shape[0]
    per_tile = E // NW
    m = per_tile // CHUNK
    n_pairs = m // 2

    mesh = plsc.VectorSubcoreMesh(core_axis_name="c", subcore_axis_name="s")

    @functools.partial(
        pl.kernel,
        mesh=mesh,
        out_type=jax.ShapeDtypeStruct((NC * ACC_ROWS, D), jnp.float32),
        scratch_types=[
            pltpu.VMEM((CHUNK,), jnp.int32),
            pltpu.VMEM((CHUNK,), jnp.int32),
            pltpu.VMEM((CHUNK,), jnp.int32),
            pltpu.VMEM((CHUNK,), jnp.int32),
            pltpu.VMEM((CHUNK, D), jnp.float32),
            pltpu.VMEM((CHUNK, D), jnp.float32),
            pltpu.VMEM_SHARED((ACC_ROWS, D), jnp.float32),
            pltpu.SemaphoreType.DMA,
            pltpu.SemaphoreType.DMA,
            pltpu.SemaphoreType.DMA,
            pltpu.SemaphoreType.DMA,
            pltpu.SemaphoreType.DMA,
        ],
    )
    def k(node_hbm, src_hbm, dst_hbm, zeros_hbm, out_hbm,
          src_a, dst_a, src_b, dst_b, rows_a, rows_b,
          acc, sem_a, sem_b, sem_s, sem_ia, sem_ib):
        c = lax.axis_index("c")
        s = lax.axis_index("s")
        wid = s * NC + c

        r0 = s * ROWS_PER_TILE
        pltpu.sync_copy(zeros_hbm.at[pl.ds(r0, ROWS_PER_TILE)],
                        acc.at[pl.ds(r0, ROWS_PER_TILE)])
        plsc.subcore_barrier()

        base0 = wid * per_tile

        def idx_start(chunk, src_c, dst_c, sem):
            e = base0 + chunk * CHUNK
            pltpu.async_copy(src_hbm.at[pl.ds(e, CHUNK)], src_c, sem)
            pltpu.async_copy(dst_hbm.at[pl.ds(e, CHUNK)], dst_c, sem)

        def idx_wait(chunk, src_c, dst_c, sem):
            e = base0 + chunk * CHUNK
            pltpu.make_async_copy(src_hbm.at[pl.ds(e, CHUNK)], src_c,
                                  sem).wait()
            pltpu.make_async_copy(dst_hbm.at[pl.ds(e, CHUNK)], dst_c,
                                  sem).wait()

        def gather_start(src_c, rows_v, sem):
            pltpu.async_copy(node_hbm.at[src_c], rows_v, sem)

        def gather_wait(src_c, rows_v, sem):
            pltpu.make_async_copy(node_hbm.at[src_c], rows_v, sem).wait()

        def scatter_start(dst_c, rows_v):
            pltpu.async_copy(rows_v, acc.at[dst_c], sem_s, add=True)

        def scatter_wait(dst_c, rows_v):
            pltpu.make_async_copy(rows_v, acc.at[dst_c], sem_s).wait()

        def scatter_sync(dst_c, rows_v):
            pltpu.sync_copy(rows_v, acc.at[dst_c], add=True)

        idx_start(0, src_a, dst_a, sem_ia)
        idx_wait(0, src_a, dst_a, sem_ia)
        gather_start(src_a, rows_a, sem_a)

        def body(i, carry):
            idx_start(2 * i + 1, src_b, dst_b, sem_ib)
            gather_wait(src_a, rows_a, sem_a)
            scatter_start(dst_a, rows_a)
            idx_wait(2 * i + 1, src_b, dst_b, sem_ib)
            gather_start(src_b, rows_b, sem_b)
            scatter_wait(dst_a, rows_a)
            idx_start(2 * i + 2, src_a, dst_a, sem_ia)
            idx_wait(2 * i + 2, src_a, dst_a, sem_ia)
            gather_start(src_a, rows_a, sem_a)
            gather_wait(src_b, rows_b, sem_b)
            scatter_sync(dst_b, rows_b)
            return carry

        lax.fori_loop(0, n_pairs, body, 0)
        gather_wait(src_a, rows_a, sem_a)
        scatter_sync(dst_a, rows_a)
        plsc.subcore_barrier()

        pltpu.sync_copy(acc.at[pl.ds(r0, ROWS_PER_TILE)],
                        out_hbm.at[pl.ds(c * ACC_ROWS + r0, ROWS_PER_TILE)])

    return k(node, src, dst, zeros)


def _combine(partials):
    R = 400

    def body(a_ref, b_ref, o_ref):
        o_ref[...] = a_ref[...] + b_ref[...]

    return pl.pallas_call(
        body,
        grid=(N_DST // R,),
        in_specs=[pl.BlockSpec((R, D), lambda i: (i, 0)),
                  pl.BlockSpec((R, D), lambda i: (i, 0))],
        out_specs=pl.BlockSpec((R, D), lambda i: (i, 0)),
        out_shape=jax.ShapeDtypeStruct((N_DST, D), jnp.float32),
    )(partials[:N_DST], partials[ACC_ROWS:ACC_ROWS + N_DST])


def kernel(node, edge_index):
    ei = edge_index.astype(jnp.int32)
    zeros = jnp.zeros((ACC_ROWS, D), jnp.float32)
    partials = _sc_partial_sums(node, ei[0], ei[1], zeros)
    return _combine(partials)
